# Initial kernel scaffold; baseline (speedup 1.0000x reference)
import functools

import jax
import jax.numpy as jnp
from jax import lax
from jax.experimental import pallas as pl
from jax.experimental.pallas import tpu as pltpu

_T = 128


def _router_body(x_ref, wg_ref, idx_ref, gate_ref):
    logits = jnp.dot(x_ref[...], wg_ref[...], preferred_element_type=jnp.float32)
    m = jnp.max(logits, axis=1, keepdims=True)
    e_iota = lax.broadcasted_iota(jnp.int32, logits.shape, 1)
    idx_ref[...] = jnp.min(
        jnp.where(logits >= m, e_iota, logits.shape[1]), axis=1, keepdims=True
    )
    gate_ref[...] = 1.0 / jnp.sum(jnp.exp(logits - m), axis=1, keepdims=True)


def _router(x, wg):
    s = x.shape[0]
    return pl.pallas_call(
        _router_body,
        out_shape=(
            jax.ShapeDtypeStruct((s, 1), jnp.int32),
            jax.ShapeDtypeStruct((s, 1), jnp.float32),
        ),
    )(x, wg)


def _moe_body(es_ref, ss_ref, lo_ref, hi_ref,
              xs_ref, gs_ref, w1_ref, b1_ref, w2_ref, b2_ref, out_ref):
    i = pl.program_id(0)

    @pl.when(i == 0)
    def _init():
        out_ref[...] = jnp.zeros_like(out_ref)

    s = ss_ref[i]
    lo = lo_ref[i]
    hi = hi_ref[i]

    @pl.when(hi > lo)
    def _compute():
        x = xs_ref[pl.ds(s, _T), :]
        rows = s + lax.broadcasted_iota(jnp.int32, (_T, 1), 0)
        msk = (rows >= lo) & (rows < hi)
        xm = jnp.where(msk, x, 0.0)
        h = jnp.maximum(
            jnp.dot(xm, w1_ref[0], preferred_element_type=jnp.float32)
            + b1_ref[...], 0.0)
        y = (jnp.dot(h, w2_ref[0], preferred_element_type=jnp.float32)
             + b2_ref[...])
        g = gs_ref[pl.ds(s, _T), :]
        out_ref[pl.ds(s, _T), :] += jnp.where(msk, y * g, 0.0)


def _group_gemm(xs, gs, w1, b1, w2, b2, es, ss, lo, hi, grid):
    s, d = xs.shape
    h = w1.shape[2]
    grid_spec = pltpu.PrefetchScalarGridSpec(
        num_scalar_prefetch=4,
        grid=(grid,),
        in_specs=[
            pl.BlockSpec((s, d), lambda i, es, ss, lo, hi: (0, 0)),
            pl.BlockSpec((s, 1), lambda i, es, ss, lo, hi: (0, 0)),
            pl.BlockSpec((1, d, h), lambda i, es, ss, lo, hi: (es[i], 0, 0)),
            pl.BlockSpec((1, h), lambda i, es, ss, lo, hi: (es[i], 0)),
            pl.BlockSpec((1, h, d), lambda i, es, ss, lo, hi: (es[i], 0, 0)),
            pl.BlockSpec((1, d), lambda i, es, ss, lo, hi: (es[i], 0)),
        ],
        out_specs=pl.BlockSpec((s, d), lambda i, es, ss, lo, hi: (0, 0)),
    )
    return pl.pallas_call(
        _moe_body,
        grid_spec=grid_spec,
        out_shape=jax.ShapeDtypeStruct((s, d), jnp.float32),
        compiler_params=pltpu.CompilerParams(
            dimension_semantics=("arbitrary",)),
    )(es, ss, lo, hi, xs, gs, w1, b1, w2, b2)


def kernel(input, Wg, W1, b1, W2, b2):
    b, s, d = input.shape
    e = W1.shape[0]
    x2d = input.reshape(s, d)

    idx2, gate2 = _router(x2d, Wg)
    top1 = idx2[:, 0]
    gate = gate2[:, 0]

    sort_idx = jnp.argsort(top1, stable=True).astype(jnp.int32)
    counts = jnp.zeros((e,), jnp.int32).at[top1].add(1)
    offsets = jnp.concatenate(
        [jnp.zeros((1,), jnp.int32), jnp.cumsum(counts, dtype=jnp.int32)])
    nblk = (counts + _T - 1) // _T
    cum_nblk = jnp.cumsum(nblk, dtype=jnp.int32)
    total = cum_nblk[-1]
    grid = s // _T + e
    g = jnp.arange(grid, dtype=jnp.int32)
    e_g = jnp.searchsorted(cum_nblk, g, side="right").astype(jnp.int32)
    valid = e_g < e
    e_c = jnp.minimum(e_g, e - 1)
    k_g = g - (cum_nblk - nblk)[e_c]
    start = offsets[e_c] + k_g * _T
    row_end = jnp.minimum(offsets[e_c + 1], start + _T)
    e_last = e_c[jnp.maximum(total - 1, 0)]
    es = jnp.where(valid, e_c, e_last)
    ss = jnp.where(valid, jnp.minimum(start, s - _T), 0)
    lo = jnp.where(valid, start, 0)
    hi = jnp.where(valid, row_end, 0)

    xs = jnp.take(x2d, sort_idx, axis=0)
    gs = jnp.take(gate, sort_idx)[:, None]

    zs = _group_gemm(xs, gs, W1, b1, W2, b2, es, ss, lo, hi, grid)

    inv = jnp.zeros((s,), jnp.int32).at[sort_idx].set(
        jnp.arange(s, dtype=jnp.int32))
    out = jnp.take(zs, inv, axis=0)
    return out.reshape(b, s, d)


# speedup vs baseline: 1.1525x; 1.1525x over previous
import functools

import jax
import jax.numpy as jnp
from jax import lax
from jax.experimental import pallas as pl
from jax.experimental.pallas import tpu as pltpu

_T = 128


def _router_body(x_ref, wg_ref, idx_ref, gate_ref):
    logits = jnp.dot(x_ref[...], wg_ref[...], preferred_element_type=jnp.float32)
    m = jnp.max(logits, axis=1, keepdims=True)
    e_iota = lax.broadcasted_iota(jnp.int32, logits.shape, 1)
    idx_ref[...] = jnp.min(
        jnp.where(logits >= m, e_iota, logits.shape[1]), axis=1, keepdims=True
    )
    gate_ref[...] = 1.0 / jnp.sum(jnp.exp(logits - m), axis=1, keepdims=True)


def _router(x, wg):
    s = x.shape[0]
    return pl.pallas_call(
        _router_body,
        out_shape=(
            jax.ShapeDtypeStruct((s, 1), jnp.int32),
            jax.ShapeDtypeStruct((s, 1), jnp.float32),
        ),
    )(x, wg)


def _moe_body(es_ref, ss_ref, lo_ref, hi_ref,
              xs_ref, gs_ref, w1_ref, b1_ref, w2_ref, b2_ref, out_ref):
    i = pl.program_id(0)

    @pl.when(i == 0)
    def _init():
        out_ref[...] = jnp.zeros_like(out_ref)

    s = pl.multiple_of(ss_ref[i], 8)
    lo = lo_ref[i]
    hi = hi_ref[i]

    @pl.when(hi > lo)
    def _compute():
        x = xs_ref[pl.ds(s, _T), :]
        rows = s + lax.broadcasted_iota(jnp.int32, (_T, 1), 0)
        msk = (rows >= lo) & (rows < hi)
        xm = jnp.where(msk, x, 0.0)
        h = jnp.maximum(
            jnp.dot(xm, w1_ref[0], preferred_element_type=jnp.float32)
            + b1_ref[0], 0.0)
        y = (jnp.dot(h, w2_ref[0], preferred_element_type=jnp.float32)
             + b2_ref[0])
        g = gs_ref[pl.ds(s, _T), :]
        out_ref[pl.ds(s, _T), :] += jnp.where(msk, y * g, 0.0)


def _group_gemm(xs, gs, w1, b1, w2, b2, es, ss, lo, hi, grid):
    s, d = xs.shape
    h = w1.shape[2]
    grid_spec = pltpu.PrefetchScalarGridSpec(
        num_scalar_prefetch=4,
        grid=(grid,),
        in_specs=[
            pl.BlockSpec((s, d), lambda i, es, ss, lo, hi: (0, 0)),
            pl.BlockSpec((s, 1), lambda i, es, ss, lo, hi: (0, 0)),
            pl.BlockSpec((1, d, h), lambda i, es, ss, lo, hi: (es[i], 0, 0)),
            pl.BlockSpec((1, 1, h), lambda i, es, ss, lo, hi: (es[i], 0, 0)),
            pl.BlockSpec((1, h, d), lambda i, es, ss, lo, hi: (es[i], 0, 0)),
            pl.BlockSpec((1, 1, d), lambda i, es, ss, lo, hi: (es[i], 0, 0)),
        ],
        out_specs=pl.BlockSpec((s, d), lambda i, es, ss, lo, hi: (0, 0)),
    )
    return pl.pallas_call(
        _moe_body,
        grid_spec=grid_spec,
        out_shape=jax.ShapeDtypeStruct((s, d), jnp.float32),
        compiler_params=pltpu.CompilerParams(
            dimension_semantics=("arbitrary",)),
    )(es, ss, lo, hi, xs, gs, w1, b1[:, None, :], w2, b2[:, None, :])


def kernel(input, Wg, W1, b1, W2, b2):
    b, s, d = input.shape
    e = W1.shape[0]
    x2d = input.reshape(s, d)

    idx2, gate2 = _router(x2d, Wg)
    top1 = idx2[:, 0]
    gate = gate2[:, 0]

    sort_idx = jnp.argsort(top1, stable=True).astype(jnp.int32)
    counts = jnp.zeros((e,), jnp.int32).at[top1].add(1)
    offsets = jnp.concatenate(
        [jnp.zeros((1,), jnp.int32), jnp.cumsum(counts, dtype=jnp.int32)])
    a8 = offsets[:-1] - (offsets[:-1] % 8)
    seg_len = offsets[1:] - a8
    nblk = jnp.where(counts > 0, (seg_len + _T - 1) // _T, 0)
    cum_nblk = jnp.cumsum(nblk, dtype=jnp.int32)
    total = cum_nblk[-1]
    grid = s // _T + e + 1
    g = jnp.arange(grid, dtype=jnp.int32)
    e_g = jnp.searchsorted(cum_nblk, g, side="right").astype(jnp.int32)
    valid = e_g < e
    e_c = jnp.minimum(e_g, e - 1)
    k_g = g - (cum_nblk - nblk)[e_c]
    start = a8[e_c] + k_g * _T
    e_last = e_c[jnp.maximum(total - 1, 0)]
    es = jnp.where(valid, e_c, e_last)
    ss = jnp.where(valid, jnp.minimum(start, s - _T), 0)
    lo = jnp.where(valid, jnp.maximum(start, offsets[e_c]), 0)
    hi = jnp.where(valid, jnp.minimum(offsets[e_c + 1], start + _T), 0)

    xs = jnp.take(x2d, sort_idx, axis=0)
    gs = jnp.take(gate, sort_idx)[:, None]

    zs = _group_gemm(xs, gs, W1, b1, W2, b2, es, ss, lo, hi, grid)

    inv = jnp.zeros((s,), jnp.int32).at[sort_idx].set(
        jnp.arange(s, dtype=jnp.int32))
    out = jnp.take(zs, inv, axis=0)
    return out.reshape(b, s, d)


# speedup vs baseline: 1.5323x; 1.3295x over previous
import functools

import jax
import jax.numpy as jnp
from jax import lax
from jax.experimental import pallas as pl
from jax.experimental.pallas import tpu as pltpu
from jax.experimental.pallas import tpu_sc as plsc

_T = 128
_HT = 512
_NW = 32


def _sc_dispatch(x3d, gate, pos):
    _, s, d = x3d.shape
    bpw = s // _NW
    gl = 128
    mesh = plsc.VectorSubcoreMesh(core_axis_name="c", subcore_axis_name="s")

    @functools.partial(
        pl.kernel, mesh=mesh,
        out_type=(
            jax.ShapeDtypeStruct((s, d), jnp.float32),
            jax.ShapeDtypeStruct((s, gl), jnp.float32),
        ),
        scratch_types=[
            pltpu.VMEM((bpw,), jnp.int32),
            pltpu.VMEM((bpw, d), jnp.float32),
            pltpu.VMEM((bpw, gl), jnp.float32),
            pltpu.SemaphoreType.DMA,
            pltpu.SemaphoreType.DMA,
        ],
    )
    def k(x_hbm, g_hbm, pos_hbm, xs_hbm, gs_hbm,
          idx_v, rows_v, grows_v, sem, sem2):
        wid = lax.axis_index("s") * 2 + lax.axis_index("c")
        base = wid * bpw
        pltpu.sync_copy(pos_hbm.at[pl.ds(base, bpw)], idx_v)
        pltpu.sync_copy(x_hbm.at[0, pl.ds(base, bpw)], rows_v)
        pltpu.sync_copy(g_hbm.at[pl.ds(base, bpw)], grows_v)
        a = pltpu.async_copy(rows_v, xs_hbm.at[idx_v], sem)
        b = pltpu.async_copy(grows_v, gs_hbm.at[idx_v], sem2)
        a.wait()
        b.wait()

    return k(x3d, gate, pos)


def _sc_combine(zs, pos):
    s, d = zs.shape
    bpw = s // _NW
    mesh = plsc.VectorSubcoreMesh(core_axis_name="c", subcore_axis_name="s")

    @functools.partial(
        pl.kernel, mesh=mesh,
        out_type=jax.ShapeDtypeStruct((1, s, d), jnp.float32),
        scratch_types=[
            pltpu.VMEM((bpw,), jnp.int32),
            pltpu.VMEM((bpw, d), jnp.float32),
            pltpu.SemaphoreType.DMA,
        ],
    )
    def k(zs_hbm, pos_hbm, out_hbm, idx_v, rows_v, sem):
        wid = lax.axis_index("s") * 2 + lax.axis_index("c")
        base = wid * bpw
        pltpu.sync_copy(pos_hbm.at[pl.ds(base, bpw)], idx_v)
        pltpu.async_copy(zs_hbm.at[idx_v], rows_v, sem).wait()
        pltpu.sync_copy(rows_v, out_hbm.at[0, pl.ds(base, bpw)])

    return k(zs, pos)


def _lane_prefix(v, inclusive):
    e = v.shape[1]
    cols = []
    for l in range(e):
        end = l + 1 if inclusive else l
        if end == 0:
            cols.append(jnp.zeros((1, 1), v.dtype))
        else:
            cols.append(jnp.sum(v[:, :end], axis=1, keepdims=True))
    return jnp.concatenate(cols, axis=1)


def _router_body(x_ref, wg_ref, pos_ref, gate_ref, sched_ref):
    s = x_ref.shape[1]
    e = wg_ref.shape[1]
    grid = sched_ref.shape[0]

    logits = jnp.dot(x_ref[0], wg_ref[...], preferred_element_type=jnp.float32)
    m = jnp.max(logits, axis=1, keepdims=True)
    e_lane = lax.broadcasted_iota(jnp.int32, (s, e), 1)
    idx = jnp.min(jnp.where(logits >= m, e_lane, e), axis=1, keepdims=True)
    gate = 1.0 / jnp.sum(jnp.exp(logits - m), axis=1, keepdims=True)
    gate_ref[...] = jnp.broadcast_to(gate, gate_ref.shape)

    oh = (e_lane == idx).astype(jnp.float32)

    ch = _T
    nch = s // ch
    r_ = lax.broadcasted_iota(jnp.int32, (ch, ch), 0)
    c_ = lax.broadcasted_iota(jnp.int32, (ch, ch), 1)
    tril = (r_ >= c_).astype(jnp.float32)
    parts = [jnp.dot(tril, oh[ci * ch:(ci + 1) * ch, :],
                     preferred_element_type=jnp.float32) for ci in range(nch)]
    chunk_sums = jnp.concatenate(
        [p[ch - 1:ch, :] for p in parts], axis=0)
    rs = lax.broadcasted_iota(jnp.int32, (nch, nch), 0)
    cs = lax.broadcasted_iota(jnp.int32, (nch, nch), 1)
    tril_x = (rs > cs).astype(jnp.float32)
    chunk_off = jnp.dot(tril_x, chunk_sums,
                        preferred_element_type=jnp.float32)
    csum = jnp.concatenate(
        [parts[ci] + chunk_off[ci:ci + 1, :] for ci in range(nch)], axis=0)

    counts = csum[s - 1:s, :].astype(jnp.int32)
    offs = _lane_prefix(counts, inclusive=False)
    offs1 = offs + counts

    rank = jnp.sum(csum * oh, axis=1, keepdims=True).astype(jnp.int32) - 1
    offtok = jnp.sum(jnp.where(oh > 0, offs.astype(jnp.float32), 0.0),
                     axis=1, keepdims=True).astype(jnp.int32)
    pos_ref[...] = offtok + rank

    a8 = offs - (offs & 7)
    seg_len = offs1 - a8
    nblk = jnp.where(counts > 0, (seg_len + _T - 1) // _T, 0)
    cum_nblk = _lane_prefix(nblk, inclusive=True)
    cum0 = cum_nblk - nblk
    total = cum_nblk[:, e - 1:e]

    g_sub = lax.broadcasted_iota(jnp.int32, (grid, 1), 0)
    e_g = jnp.sum((jnp.broadcast_to(cum_nblk, (grid, e)) <= g_sub)
                  .astype(jnp.int32), axis=1, keepdims=True)
    e_c = jnp.minimum(e_g, e - 1)
    ohg = (lax.broadcasted_iota(jnp.int32, (grid, e), 1) == e_c)

    def lookup(v):
        return jnp.sum(jnp.where(ohg, jnp.broadcast_to(v, (grid, e)), 0),
                       axis=1, keepdims=True)

    k_g = g_sub - lookup(cum0)
    start = lookup(a8) + k_g * _T
    lo = jnp.maximum(start, lookup(offs))
    hi = jnp.minimum(lookup(offs1), start + _T)
    valid = g_sub < total
    e_last = jnp.sum(jnp.where(g_sub == total - 1, e_c, 0),
                     axis=0, keepdims=True)
    es = jnp.where(valid, e_c, jnp.broadcast_to(e_last, (grid, 1)))
    ss = jnp.where(valid, jnp.minimum(start, s - _T), 0)
    lo = jnp.where(valid, lo, 0)
    hi = jnp.where(valid, hi, 0)
    sched_ref[...] = jnp.concatenate([es, ss, lo, hi], axis=1)


def _router(x, wg, grid):
    s = x.shape[1]
    return pl.pallas_call(
        _router_body,
        out_shape=(
            jax.ShapeDtypeStruct((s, 1), jnp.int32),
            jax.ShapeDtypeStruct((s, 128), jnp.float32),
            jax.ShapeDtypeStruct((grid, 4), jnp.int32),
        ),
    )(x, wg)


def _moe_body(es_ref, ss_ref, lo_ref, hi_ref,
              xs_ref, gs_ref, w1_ref, b1_ref, w2_ref, b2_ref, out_ref):
    i = pl.program_id(0)

    @pl.when(i == 0)
    def _init():
        out_ref[...] = jnp.zeros_like(out_ref)

    s = pl.multiple_of(ss_ref[i], 8)
    lo = lo_ref[i]
    hi = hi_ref[i]

    @pl.when(hi > lo)
    def _compute():
        x = xs_ref[pl.ds(s, _T), :]
        rows = s + lax.broadcasted_iota(jnp.int32, (_T, 1), 0)
        msk = (rows >= lo) & (rows < hi)
        xm = jnp.where(msk, x, 0.0)
        h = jnp.maximum(
            jnp.dot(xm, w1_ref[0], preferred_element_type=jnp.float32)
            + b1_ref[0], 0.0)
        y = (jnp.dot(h, w2_ref[0], preferred_element_type=jnp.float32)
             + b2_ref[0])
        g = gs_ref[pl.ds(s, _T), 0:1]
        out_ref[pl.ds(s, _T), :] += jnp.where(msk, y * g, 0.0)


def _group_gemm(xs, gs, w1, b1, w2, b2, es, ss, lo, hi, grid):
    s, d = xs.shape
    h = w1.shape[2]
    grid_spec = pltpu.PrefetchScalarGridSpec(
        num_scalar_prefetch=4,
        grid=(grid,),
        in_specs=[
            pl.BlockSpec((s, d), lambda i, es, ss, lo, hi: (0, 0)),
            pl.BlockSpec((s, 128), lambda i, es, ss, lo, hi: (0, 0)),
            pl.BlockSpec((1, d, h), lambda i, es, ss, lo, hi: (es[i], 0, 0)),
            pl.BlockSpec((1, 1, h), lambda i, es, ss, lo, hi: (es[i], 0, 0)),
            pl.BlockSpec((1, h, d), lambda i, es, ss, lo, hi: (es[i], 0, 0)),
            pl.BlockSpec((1, 1, d), lambda i, es, ss, lo, hi: (es[i], 0, 0)),
        ],
        out_specs=pl.BlockSpec((s, d), lambda i, es, ss, lo, hi: (0, 0)),
    )
    return pl.pallas_call(
        _moe_body,
        grid_spec=grid_spec,
        out_shape=jax.ShapeDtypeStruct((s, d), jnp.float32),
        compiler_params=pltpu.CompilerParams(
            dimension_semantics=("arbitrary",)),
    )(es, ss, lo, hi, xs, gs, w1, b1[:, None, :], w2, b2[:, None, :])


def kernel(input, Wg, W1, b1, W2, b2):
    b, s, d = input.shape
    e = W1.shape[0]
    grid = s // _T + e + 1

    pos2, gate2, sched = _router(input, Wg, grid)
    pos = pos2[:, 0]
    es, ss, lo, hi = sched[:, 0], sched[:, 1], sched[:, 2], sched[:, 3]

    xs, gs = _sc_dispatch(input, gate2, pos)

    zs = _group_gemm(xs, gs, W1, b1, W2, b2, es, ss, lo, hi, grid)

    return _sc_combine(zs, pos)


# speedup vs baseline: 1.6454x; 1.0738x over previous
import functools

import jax
import jax.numpy as jnp
from jax import lax
from jax.experimental import pallas as pl
from jax.experimental.pallas import tpu as pltpu
from jax.experimental.pallas import tpu_sc as plsc

_T = 256
_HT = 512
_NW = 32


def _sc_dispatch(x3d, gate, pos):
    _, s, d = x3d.shape
    bpw = s // _NW
    gl = 128
    mesh = plsc.VectorSubcoreMesh(core_axis_name="c", subcore_axis_name="s")

    @functools.partial(
        pl.kernel, mesh=mesh,
        out_type=(
            jax.ShapeDtypeStruct((s, d), jnp.float32),
            jax.ShapeDtypeStruct((s, gl), jnp.float32),
        ),
        scratch_types=[
            pltpu.VMEM((bpw,), jnp.int32),
            pltpu.VMEM((bpw, d), jnp.float32),
            pltpu.VMEM((bpw, gl), jnp.float32),
            pltpu.SemaphoreType.DMA,
            pltpu.SemaphoreType.DMA,
        ],
    )
    def k(x_hbm, g_hbm, pos_hbm, xs_hbm, gs_hbm,
          idx_v, rows_v, grows_v, sem, sem2):
        wid = lax.axis_index("s") * 2 + lax.axis_index("c")
        base = wid * bpw
        pltpu.sync_copy(pos_hbm.at[pl.ds(base, bpw)], idx_v)
        pltpu.sync_copy(x_hbm.at[0, pl.ds(base, bpw)], rows_v)
        pltpu.sync_copy(g_hbm.at[pl.ds(base, bpw)], grows_v)
        a = pltpu.async_copy(rows_v, xs_hbm.at[idx_v], sem)
        b = pltpu.async_copy(grows_v, gs_hbm.at[idx_v], sem2)
        a.wait()
        b.wait()

    return k(x3d, gate, pos)


def _sc_combine(zs, pos):
    s, d = zs.shape
    bpw = s // _NW
    mesh = plsc.VectorSubcoreMesh(core_axis_name="c", subcore_axis_name="s")

    @functools.partial(
        pl.kernel, mesh=mesh,
        out_type=jax.ShapeDtypeStruct((1, s, d), jnp.float32),
        scratch_types=[
            pltpu.VMEM((bpw,), jnp.int32),
            pltpu.VMEM((bpw, d), jnp.float32),
            pltpu.SemaphoreType.DMA,
        ],
    )
    def k(zs_hbm, pos_hbm, out_hbm, idx_v, rows_v, sem):
        wid = lax.axis_index("s") * 2 + lax.axis_index("c")
        base = wid * bpw
        pltpu.sync_copy(pos_hbm.at[pl.ds(base, bpw)], idx_v)
        pltpu.async_copy(zs_hbm.at[idx_v], rows_v, sem).wait()
        pltpu.sync_copy(rows_v, out_hbm.at[0, pl.ds(base, bpw)])

    return k(zs, pos)


def _lane_prefix(v, inclusive):
    e = v.shape[1]
    cols = []
    for l in range(e):
        end = l + 1 if inclusive else l
        if end == 0:
            cols.append(jnp.zeros((1, 1), v.dtype))
        else:
            cols.append(jnp.sum(v[:, :end], axis=1, keepdims=True))
    return jnp.concatenate(cols, axis=1)


def _router_body(x_ref, wg_ref, pos_ref, gate_ref, sched_ref):
    s = x_ref.shape[1]
    e = wg_ref.shape[1]
    grid = sched_ref.shape[0]

    logits = jnp.dot(x_ref[0], wg_ref[...], preferred_element_type=jnp.float32)
    m = jnp.max(logits, axis=1, keepdims=True)
    e_lane = lax.broadcasted_iota(jnp.int32, (s, e), 1)
    idx = jnp.min(jnp.where(logits >= m, e_lane, e), axis=1, keepdims=True)
    gate = 1.0 / jnp.sum(jnp.exp(logits - m), axis=1, keepdims=True)
    gate_ref[...] = jnp.broadcast_to(gate, gate_ref.shape)

    oh = (e_lane == idx).astype(jnp.float32)

    ch = _T
    nch = s // ch
    r_ = lax.broadcasted_iota(jnp.int32, (ch, ch), 0)
    c_ = lax.broadcasted_iota(jnp.int32, (ch, ch), 1)
    tril = (r_ >= c_).astype(jnp.float32)
    parts = [jnp.dot(tril, oh[ci * ch:(ci + 1) * ch, :],
                     preferred_element_type=jnp.float32) for ci in range(nch)]
    chunk_sums = jnp.concatenate(
        [p[ch - 1:ch, :] for p in parts], axis=0)
    rs = lax.broadcasted_iota(jnp.int32, (nch, nch), 0)
    cs = lax.broadcasted_iota(jnp.int32, (nch, nch), 1)
    tril_x = (rs > cs).astype(jnp.float32)
    chunk_off = jnp.dot(tril_x, chunk_sums,
                        preferred_element_type=jnp.float32)
    csum = jnp.concatenate(
        [parts[ci] + chunk_off[ci:ci + 1, :] for ci in range(nch)], axis=0)

    counts = csum[s - 1:s, :].astype(jnp.int32)
    offs = _lane_prefix(counts, inclusive=False)
    offs1 = offs + counts

    rank = jnp.sum(csum * oh, axis=1, keepdims=True).astype(jnp.int32) - 1
    offtok = jnp.sum(jnp.where(oh > 0, offs.astype(jnp.float32), 0.0),
                     axis=1, keepdims=True).astype(jnp.int32)
    pos_ref[...] = offtok + rank

    a8 = offs - (offs & 7)
    seg_len = offs1 - a8
    nblk = jnp.where(counts > 0, (seg_len + _T - 1) // _T, 0)
    cum_nblk = _lane_prefix(nblk, inclusive=True)
    cum0 = cum_nblk - nblk
    total = cum_nblk[:, e - 1:e]

    g_sub = lax.broadcasted_iota(jnp.int32, (grid, 1), 0)
    e_g = jnp.sum((jnp.broadcast_to(cum_nblk, (grid, e)) <= g_sub)
                  .astype(jnp.int32), axis=1, keepdims=True)
    e_c = jnp.minimum(e_g, e - 1)
    ohg = (lax.broadcasted_iota(jnp.int32, (grid, e), 1) == e_c)

    def lookup(v):
        return jnp.sum(jnp.where(ohg, jnp.broadcast_to(v, (grid, e)), 0),
                       axis=1, keepdims=True)

    k_g = g_sub - lookup(cum0)
    start = lookup(a8) + k_g * _T
    lo = jnp.maximum(start, lookup(offs))
    hi = jnp.minimum(lookup(offs1), start + _T)
    valid = g_sub < total
    e_last = jnp.sum(jnp.where(g_sub == total - 1, e_c, 0),
                     axis=0, keepdims=True)
    es = jnp.where(valid, e_c, jnp.broadcast_to(e_last, (grid, 1)))
    ss = jnp.where(valid, jnp.minimum(start, s - _T), 0)
    lo = jnp.where(valid, lo, 0)
    hi = jnp.where(valid, hi, 0)
    sched_ref[...] = jnp.concatenate([es, ss, lo, hi], axis=1)


def _router(x, wg, grid):
    s = x.shape[1]
    return pl.pallas_call(
        _router_body,
        out_shape=(
            jax.ShapeDtypeStruct((s, 1), jnp.int32),
            jax.ShapeDtypeStruct((s, 128), jnp.float32),
            jax.ShapeDtypeStruct((grid, 4), jnp.int32),
        ),
    )(x, wg)


def _moe_body(es_ref, ss_ref, lo_ref, hi_ref,
              xs_ref, gs_ref, w1_ref, b1_ref, w2_ref, b2_ref, out_ref):
    i = pl.program_id(0)

    @pl.when(i == 0)
    def _init():
        out_ref[...] = jnp.zeros_like(out_ref)

    s = pl.multiple_of(ss_ref[i], 8)
    lo = lo_ref[i]
    hi = hi_ref[i]

    @pl.when(hi > lo)
    def _compute():
        x = xs_ref[pl.ds(s, _T), :]
        rows = s + lax.broadcasted_iota(jnp.int32, (_T, 1), 0)
        msk = (rows >= lo) & (rows < hi)
        xm = jnp.where(msk, x, 0.0)
        h = jnp.maximum(
            jnp.dot(xm, w1_ref[0], preferred_element_type=jnp.float32)
            + b1_ref[0], 0.0)
        y = (jnp.dot(h, w2_ref[0], preferred_element_type=jnp.float32)
             + b2_ref[0])
        g = gs_ref[pl.ds(s, _T), 0:1]
        out_ref[pl.ds(s, _T), :] += jnp.where(msk, y * g, 0.0)


def _group_gemm(xs, gs, w1, b1, w2, b2, es, ss, lo, hi, grid):
    s, d = xs.shape
    h = w1.shape[2]
    grid_spec = pltpu.PrefetchScalarGridSpec(
        num_scalar_prefetch=4,
        grid=(grid,),
        in_specs=[
            pl.BlockSpec((s, d), lambda i, es, ss, lo, hi: (0, 0)),
            pl.BlockSpec((s, 128), lambda i, es, ss, lo, hi: (0, 0)),
            pl.BlockSpec((1, d, h), lambda i, es, ss, lo, hi: (es[i], 0, 0)),
            pl.BlockSpec((1, 1, h), lambda i, es, ss, lo, hi: (es[i], 0, 0)),
            pl.BlockSpec((1, h, d), lambda i, es, ss, lo, hi: (es[i], 0, 0)),
            pl.BlockSpec((1, 1, d), lambda i, es, ss, lo, hi: (es[i], 0, 0)),
        ],
        out_specs=pl.BlockSpec((s, d), lambda i, es, ss, lo, hi: (0, 0)),
    )
    return pl.pallas_call(
        _moe_body,
        grid_spec=grid_spec,
        out_shape=jax.ShapeDtypeStruct((s, d), jnp.float32),
        compiler_params=pltpu.CompilerParams(
            dimension_semantics=("arbitrary",)),
    )(es, ss, lo, hi, xs, gs, w1, b1[:, None, :], w2, b2[:, None, :])


def kernel(input, Wg, W1, b1, W2, b2):
    b, s, d = input.shape
    e = W1.shape[0]
    grid = s // _T + e + 1

    pos2, gate2, sched = _router(input, Wg, grid)
    pos = pos2[:, 0]
    es, ss, lo, hi = sched[:, 0], sched[:, 1], sched[:, 2], sched[:, 3]

    xs, gs = _sc_dispatch(input, gate2, pos)

    zs = _group_gemm(xs, gs, W1, b1, W2, b2, es, ss, lo, hi, grid)

    return _sc_combine(zs, pos)


# speedup vs baseline: 1.6756x; 1.0184x over previous
import functools

import jax
import jax.numpy as jnp
from jax import lax
from jax.experimental import pallas as pl
from jax.experimental.pallas import tpu as pltpu
from jax.experimental.pallas import tpu_sc as plsc

_T = 256
_HT = 512
_NW = 32


def _sc_dispatch(x3d, gate, pos):
    _, s, d = x3d.shape
    bpw = s // _NW
    gl = 128
    mesh = plsc.VectorSubcoreMesh(core_axis_name="c", subcore_axis_name="s")

    @functools.partial(
        pl.kernel, mesh=mesh,
        out_type=(
            jax.ShapeDtypeStruct((s, d), jnp.float32),
            jax.ShapeDtypeStruct((s, gl), jnp.float32),
        ),
        scratch_types=[
            pltpu.VMEM((bpw,), jnp.int32),
            pltpu.VMEM((bpw, d), jnp.float32),
            pltpu.VMEM((bpw, gl), jnp.float32),
            pltpu.SemaphoreType.DMA,
            pltpu.SemaphoreType.DMA,
        ],
    )
    def k(x_hbm, g_hbm, pos_hbm, xs_hbm, gs_hbm,
          idx_v, rows_v, grows_v, sem, sem2):
        wid = lax.axis_index("s") * 2 + lax.axis_index("c")
        base = wid * bpw
        pltpu.sync_copy(pos_hbm.at[pl.ds(base, bpw)], idx_v)
        pltpu.sync_copy(x_hbm.at[0, pl.ds(base, bpw)], rows_v)
        pltpu.sync_copy(g_hbm.at[pl.ds(base, bpw)], grows_v)
        a = pltpu.async_copy(rows_v, xs_hbm.at[idx_v], sem)
        b = pltpu.async_copy(grows_v, gs_hbm.at[idx_v], sem2)
        a.wait()
        b.wait()

    return k(x3d, gate, pos)


def _sc_combine(zs, pos):
    s, d = zs.shape
    bpw = s // _NW
    mesh = plsc.VectorSubcoreMesh(core_axis_name="c", subcore_axis_name="s")

    @functools.partial(
        pl.kernel, mesh=mesh,
        out_type=jax.ShapeDtypeStruct((1, s, d), jnp.float32),
        scratch_types=[
            pltpu.VMEM((bpw,), jnp.int32),
            pltpu.VMEM((bpw, d), jnp.float32),
            pltpu.SemaphoreType.DMA,
        ],
    )
    def k(zs_hbm, pos_hbm, out_hbm, idx_v, rows_v, sem):
        wid = lax.axis_index("s") * 2 + lax.axis_index("c")
        base = wid * bpw
        pltpu.sync_copy(pos_hbm.at[pl.ds(base, bpw)], idx_v)
        pltpu.async_copy(zs_hbm.at[idx_v], rows_v, sem).wait()
        pltpu.sync_copy(rows_v, out_hbm.at[0, pl.ds(base, bpw)])

    return k(zs, pos)


def _lane_prefix(v, inclusive):
    e = v.shape[1]
    cols = []
    for l in range(e):
        end = l + 1 if inclusive else l
        if end == 0:
            cols.append(jnp.zeros((1, 1), v.dtype))
        else:
            cols.append(jnp.sum(v[:, :end], axis=1, keepdims=True))
    return jnp.concatenate(cols, axis=1)


def _router_body(x_ref, wg_ref, pos_ref, gate_ref,
                 es_ref, ss_ref, lo_ref, hi_ref):
    s = x_ref.shape[1]
    e = wg_ref.shape[1]
    grid = es_ref.shape[0]

    logits = jnp.dot(x_ref[0], wg_ref[...], preferred_element_type=jnp.float32)
    m = jnp.max(logits, axis=1, keepdims=True)
    e_lane = lax.broadcasted_iota(jnp.int32, (s, e), 1)
    idx = jnp.min(jnp.where(logits >= m, e_lane, e), axis=1, keepdims=True)
    gate = 1.0 / jnp.sum(jnp.exp(logits - m), axis=1, keepdims=True)
    gate_ref[...] = jnp.broadcast_to(gate, gate_ref.shape)

    oh = (e_lane == idx).astype(jnp.float32)

    ch = _T
    nch = s // ch
    r_ = lax.broadcasted_iota(jnp.int32, (ch, ch), 0)
    c_ = lax.broadcasted_iota(jnp.int32, (ch, ch), 1)
    tril = (r_ >= c_).astype(jnp.float32)
    parts = [jnp.dot(tril, oh[ci * ch:(ci + 1) * ch, :],
                     preferred_element_type=jnp.float32) for ci in range(nch)]
    chunk_sums = jnp.concatenate(
        [p[ch - 1:ch, :] for p in parts], axis=0)
    rs = lax.broadcasted_iota(jnp.int32, (nch, nch), 0)
    cs = lax.broadcasted_iota(jnp.int32, (nch, nch), 1)
    tril_x = (rs > cs).astype(jnp.float32)
    chunk_off = jnp.dot(tril_x, chunk_sums,
                        preferred_element_type=jnp.float32)
    csum = jnp.concatenate(
        [parts[ci] + chunk_off[ci:ci + 1, :] for ci in range(nch)], axis=0)

    counts = csum[s - 1:s, :].astype(jnp.int32)
    offs = _lane_prefix(counts, inclusive=False)
    offs1 = offs + counts

    rank = jnp.sum(csum * oh, axis=1, keepdims=True).astype(jnp.int32) - 1
    offtok = jnp.sum(jnp.where(oh > 0, offs.astype(jnp.float32), 0.0),
                     axis=1, keepdims=True).astype(jnp.int32)
    pos_ref[...] = (offtok + rank).reshape(s)

    a8 = offs - (offs & 7)
    seg_len = offs1 - a8
    nblk = jnp.where(counts > 0, (seg_len + _T - 1) // _T, 0)
    cum_nblk = _lane_prefix(nblk, inclusive=True)
    cum0 = cum_nblk - nblk
    total = cum_nblk[:, e - 1:e]

    g_sub = lax.broadcasted_iota(jnp.int32, (grid, 1), 0)
    e_g = jnp.sum((jnp.broadcast_to(cum_nblk, (grid, e)) <= g_sub)
                  .astype(jnp.int32), axis=1, keepdims=True)
    e_c = jnp.minimum(e_g, e - 1)
    ohg = (lax.broadcasted_iota(jnp.int32, (grid, e), 1) == e_c)

    def lookup(v):
        return jnp.sum(jnp.where(ohg, jnp.broadcast_to(v, (grid, e)), 0),
                       axis=1, keepdims=True)

    k_g = g_sub - lookup(cum0)
    start = lookup(a8) + k_g * _T
    lo = jnp.maximum(start, lookup(offs))
    hi = jnp.minimum(lookup(offs1), start + _T)
    valid = g_sub < total
    e_last = jnp.sum(jnp.where(g_sub == total - 1, e_c, 0),
                     axis=0, keepdims=True)
    es_ref[...] = jnp.where(
        valid, e_c, jnp.broadcast_to(e_last, (grid, 1))).reshape(grid)
    ss_ref[...] = jnp.where(valid, jnp.minimum(start, s - _T), 0).reshape(grid)
    lo_ref[...] = jnp.where(valid, lo, 0).reshape(grid)
    hi_ref[...] = jnp.where(valid, hi, 0).reshape(grid)


def _router(x, wg, grid):
    s = x.shape[1]
    return pl.pallas_call(
        _router_body,
        out_shape=(
            jax.ShapeDtypeStruct((s,), jnp.int32),
            jax.ShapeDtypeStruct((s, 128), jnp.float32),
            jax.ShapeDtypeStruct((grid,), jnp.int32),
            jax.ShapeDtypeStruct((grid,), jnp.int32),
            jax.ShapeDtypeStruct((grid,), jnp.int32),
            jax.ShapeDtypeStruct((grid,), jnp.int32),
        ),
    )(x, wg)


def _moe_body(es_ref, ss_ref, lo_ref, hi_ref,
              xs_ref, gs_ref, w1_ref, b1_ref, w2_ref, b2_ref, out_ref):
    i = pl.program_id(0)

    @pl.when(i == 0)
    def _init():
        out_ref[...] = jnp.zeros_like(out_ref)

    s = pl.multiple_of(ss_ref[i], 8)
    lo = lo_ref[i]
    hi = hi_ref[i]

    @pl.when(hi > lo)
    def _compute():
        x = xs_ref[pl.ds(s, _T), :]
        rows = s + lax.broadcasted_iota(jnp.int32, (_T, 1), 0)
        msk = (rows >= lo) & (rows < hi)
        xm = jnp.where(msk, x, 0.0)
        h = jnp.maximum(
            jnp.dot(xm, w1_ref[0], preferred_element_type=jnp.float32)
            + b1_ref[0], 0.0)
        y = (jnp.dot(h, w2_ref[0], preferred_element_type=jnp.float32)
             + b2_ref[0])
        g = gs_ref[pl.ds(s, _T), 0:1]
        out_ref[pl.ds(s, _T), :] += jnp.where(msk, y * g, 0.0)


def _group_gemm(xs, gs, w1, b1, w2, b2, es, ss, lo, hi, grid):
    s, d = xs.shape
    h = w1.shape[2]
    grid_spec = pltpu.PrefetchScalarGridSpec(
        num_scalar_prefetch=4,
        grid=(grid,),
        in_specs=[
            pl.BlockSpec((s, d), lambda i, es, ss, lo, hi: (0, 0)),
            pl.BlockSpec((s, 128), lambda i, es, ss, lo, hi: (0, 0)),
            pl.BlockSpec((1, d, h), lambda i, es, ss, lo, hi: (es[i], 0, 0)),
            pl.BlockSpec((1, 1, h), lambda i, es, ss, lo, hi: (es[i], 0, 0)),
            pl.BlockSpec((1, h, d), lambda i, es, ss, lo, hi: (es[i], 0, 0)),
            pl.BlockSpec((1, 1, d), lambda i, es, ss, lo, hi: (es[i], 0, 0)),
        ],
        out_specs=pl.BlockSpec((s, d), lambda i, es, ss, lo, hi: (0, 0)),
    )
    return pl.pallas_call(
        _moe_body,
        grid_spec=grid_spec,
        out_shape=jax.ShapeDtypeStruct((s, d), jnp.float32),
        compiler_params=pltpu.CompilerParams(
            dimension_semantics=("arbitrary",)),
    )(es, ss, lo, hi, xs, gs, w1, b1[:, None, :], w2, b2[:, None, :])


def kernel(input, Wg, W1, b1, W2, b2):
    b, s, d = input.shape
    e = W1.shape[0]
    grid = s // _T + e + 1

    pos, gate2, es, ss, lo, hi = _router(input, Wg, grid)

    xs, gs = _sc_dispatch(input, gate2, pos)

    zs = _group_gemm(xs, gs, W1, b1, W2, b2, es, ss, lo, hi, grid)

    return _sc_combine(zs, pos)
